# baseline (speedup 1.0000x reference)
import functools

import jax
import jax.numpy as jnp
from jax import lax
from jax.experimental import pallas as pl
from jax.experimental.pallas import tpu as pltpu
from jax.experimental.pallas import tpu_sc as plsc

_R = float(1.25**2)
_ROWS, _COLS = 4096, 512
_NC, _NS, _L = 2, 16, 16
_NW = _NC * _NS

_S = 1024
_SROWS_W = _S // _NW
_NACC = 4

_TROWS = _ROWS - _S
_BROWS = 512
_GRID = _TROWS // _BROWS

_mesh = plsc.VectorSubcoreMesh(core_axis_name="c", subcore_axis_name="s")


@functools.partial(
    pl.kernel,
    out_type=jax.ShapeDtypeStruct((2, _NW * _L), jnp.float32),
    mesh=_mesh,
    scratch_types=[
        pltpu.VMEM((_SROWS_W, _COLS), jnp.float32),
        pltpu.VMEM((_SROWS_W, _COLS), jnp.float32),
        pltpu.VMEM((_L,), jnp.float32),
        pltpu.VMEM((_L,), jnp.float32),
        pltpu.SemaphoreType.DMA,
    ],
)
def _sc_count(pred_hbm, targ_hbm, out_hbm, pbuf, tbuf, gbuf, nbuf, sem):
    c = lax.axis_index("c")
    s = lax.axis_index("s")
    wid = s * _NC + c
    row0 = wid * _SROWS_W

    cp = pltpu.async_copy(pred_hbm.at[pl.ds(row0, _SROWS_W), :], pbuf, sem)
    ct = pltpu.async_copy(targ_hbm.at[pl.ds(row0, _SROWS_W), :], tbuf, sem)
    cp.wait()
    ct.wait()

    zero = jnp.zeros((_L,), jnp.float32)
    one = jnp.full((_L,), 1.0, jnp.float32)
    accs = (zero,) * (2 * _NACC)
    n_el = _SROWS_W * _COLS

    @plsc.parallel_loop(0, n_el, _NACC * _L, unroll=2, carry=accs)
    def accs(i, acc):
        row = i // _COLS
        col = i % _COLS
        out = list(acc)
        for j in range(_NACC):
            p = pbuf[row, pl.ds(col + j * _L, _L)]
            t = tbuf[row, pl.ds(col + j * _L, _L)]
            good = (p < _R * t) & (t < _R * p)
            out[j] = acc[j] + jnp.where(good, one, zero)
            out[_NACC + j] = acc[_NACC + j] + jnp.where(t > 0.0, one, zero)
        return tuple(out)

    acc_g = accs[0]
    acc_n = accs[_NACC]
    for j in range(1, _NACC):
        acc_g = acc_g + accs[j]
        acc_n = acc_n + accs[_NACC + j]

    gbuf[...] = acc_g
    nbuf[...] = acc_n
    pltpu.sync_copy(gbuf, out_hbm.at[0, pl.ds(wid * _L, _L)])
    pltpu.sync_copy(nbuf, out_hbm.at[1, pl.ds(wid * _L, _L)])


def _tc_body(p_ref, t_ref, out_ref):
    @pl.when(pl.program_id(0) == 0)
    def _():
        out_ref[0] = 0.0
        out_ref[1] = 0.0

    p = p_ref[...]
    t = t_ref[...]
    good = (p < _R * t) & (t < _R * p)
    out_ref[0] += jnp.sum(good.astype(jnp.float32))
    out_ref[1] += jnp.sum((t > 0.0).astype(jnp.float32))


_tc_count = pl.pallas_call(
    _tc_body,
    grid=(_GRID,),
    in_specs=[
        pl.BlockSpec((_BROWS, _COLS), lambda i: (i + _S // _BROWS, 0)),
        pl.BlockSpec((_BROWS, _COLS), lambda i: (i + _S // _BROWS, 0)),
    ],
    out_specs=pl.BlockSpec(memory_space=pltpu.SMEM),
    out_shape=jax.ShapeDtypeStruct((2,), jnp.float32),
    compiler_params=pltpu.CompilerParams(
        dimension_semantics=("arbitrary",),
    ),
)


def kernel(pred, target):
    p = pred.reshape(_ROWS, _COLS)
    t = target.reshape(_ROWS, _COLS)
    sc_parts = _sc_count(p, t)
    tc_counts = _tc_count(p, t)
    good = sc_parts[0].sum() + tc_counts[0]
    npix = sc_parts[1].sum() + tc_counts[1]
    return good / npix


# speedup vs baseline: 3.7239x; 3.7239x over previous
import jax
import jax.numpy as jnp
from jax.experimental import pallas as pl
from jax.experimental.pallas import tpu as pltpu

_R = float(1.25**2)
_ROWS, _COLS = 4096, 512
_BROWS = 1024
_GRID = _ROWS // _BROWS


def _tc_body(p_ref, t_ref, out_ref, acc_ref):
    @pl.when(pl.program_id(0) == 0)
    def _():
        acc_ref[0] = 0.0
        acc_ref[1] = 0.0

    p = p_ref[...]
    t = t_ref[...]
    good = (p < _R * t) & (t < _R * p)
    acc_ref[0] += jnp.sum(good.astype(jnp.float32))
    acc_ref[1] += jnp.sum((t > 0.0).astype(jnp.float32))

    @pl.when(pl.program_id(0) == _GRID - 1)
    def _():
        out_ref[0] = acc_ref[0] / acc_ref[1]


_tc_ratio = pl.pallas_call(
    _tc_body,
    grid=(_GRID,),
    in_specs=[
        pl.BlockSpec((_BROWS, _COLS), lambda i: (i, 0)),
        pl.BlockSpec((_BROWS, _COLS), lambda i: (i, 0)),
    ],
    out_specs=pl.BlockSpec(memory_space=pltpu.SMEM),
    out_shape=jax.ShapeDtypeStruct((1,), jnp.float32),
    scratch_shapes=[pltpu.SMEM((2,), jnp.float32)],
    compiler_params=pltpu.CompilerParams(
        dimension_semantics=("arbitrary",),
    ),
)


def kernel(pred, target):
    p = pred.reshape(_ROWS, _COLS)
    t = target.reshape(_ROWS, _COLS)
    return _tc_ratio(p, t)[0]


# speedup vs baseline: 3.7273x; 1.0009x over previous
import jax
import jax.numpy as jnp
from jax.experimental import pallas as pl
from jax.experimental.pallas import tpu as pltpu

_R = float(1.25**2)
_ROWS, _COLS = 4096, 512
_BROWS = 2048
_GRID = _ROWS // _BROWS


def _tc_body(p_ref, t_ref, out_ref, acc_ref):
    @pl.when(pl.program_id(0) == 0)
    def _():
        acc_ref[0] = 0.0
        acc_ref[1] = 0.0

    p = p_ref[...]
    t = t_ref[...]
    good = (p < _R * t) & (t < _R * p)
    acc_ref[0] += jnp.sum(good.astype(jnp.float32))
    acc_ref[1] += jnp.sum((t > 0.0).astype(jnp.float32))

    @pl.when(pl.program_id(0) == _GRID - 1)
    def _():
        out_ref[0] = acc_ref[0] / acc_ref[1]


_tc_ratio = pl.pallas_call(
    _tc_body,
    grid=(_GRID,),
    in_specs=[
        pl.BlockSpec((_BROWS, _COLS), lambda i: (i, 0)),
        pl.BlockSpec((_BROWS, _COLS), lambda i: (i, 0)),
    ],
    out_specs=pl.BlockSpec(memory_space=pltpu.SMEM),
    out_shape=jax.ShapeDtypeStruct((1,), jnp.float32),
    scratch_shapes=[pltpu.SMEM((2,), jnp.float32)],
    compiler_params=pltpu.CompilerParams(
        dimension_semantics=("arbitrary",),
    ),
)


def kernel(pred, target):
    p = pred.reshape(_ROWS, _COLS)
    t = target.reshape(_ROWS, _COLS)
    return _tc_ratio(p, t)[0]


# speedup vs baseline: 3.7519x; 1.0066x over previous
import jax
import jax.numpy as jnp
from jax.experimental import pallas as pl
from jax.experimental.pallas import tpu as pltpu

_R = float(1.25**2)
_ROWS, _COLS = 4096, 512
_BROWS = 1024
_HALF = _ROWS // 2
_GRID = _HALF // _BROWS


def _tc_body(pa_ref, pb_ref, ta_ref, tb_ref, out_ref, acc_ref):
    @pl.when(pl.program_id(0) == 0)
    def _():
        acc_ref[0] = 0.0
        acc_ref[1] = 0.0

    g = jnp.float32(0.0)
    n = jnp.float32(0.0)
    for p_ref, t_ref in ((pa_ref, ta_ref), (pb_ref, tb_ref)):
        p = p_ref[...]
        t = t_ref[...]
        good = (p < _R * t) & (t < _R * p)
        g += jnp.sum(good.astype(jnp.float32))
        n += jnp.sum((t > 0.0).astype(jnp.float32))
    acc_ref[0] += g
    acc_ref[1] += n

    @pl.when(pl.program_id(0) == _GRID - 1)
    def _():
        out_ref[0] = acc_ref[0] / acc_ref[1]


_tc_ratio = pl.pallas_call(
    _tc_body,
    grid=(_GRID,),
    in_specs=[
        pl.BlockSpec((_BROWS, _COLS), lambda i: (i, 0)),
        pl.BlockSpec((_BROWS, _COLS), lambda i: (i + _HALF // _BROWS, 0)),
        pl.BlockSpec((_BROWS, _COLS), lambda i: (i, 0)),
        pl.BlockSpec((_BROWS, _COLS), lambda i: (i + _HALF // _BROWS, 0)),
    ],
    out_specs=pl.BlockSpec(memory_space=pltpu.SMEM),
    out_shape=jax.ShapeDtypeStruct((1,), jnp.float32),
    scratch_shapes=[pltpu.SMEM((2,), jnp.float32)],
    compiler_params=pltpu.CompilerParams(
        dimension_semantics=("arbitrary",),
    ),
)


def kernel(pred, target):
    p = pred.reshape(_ROWS, _COLS)
    t = target.reshape(_ROWS, _COLS)
    return _tc_ratio(p, p, t, t)[0]
